# baseline (speedup 1.0000x reference)
import functools

import jax
import jax.numpy as jnp
from jax import lax
from jax.experimental import pallas as pl
from jax.experimental.pallas import tpu as pltpu, tpu_sc as plsc

NUM_LABELS = 33
EMB = 512
MAX_LEN = 2048
BATCH = 16
TOTAL = 16384

ROWS_PER_W = 1024
CHUNK = 64
NCHUNK = ROWS_PER_W // CHUNK
NBUF = 2
TOK_BUF = 1032
PAD_ID = NUM_LABELS
TAB_ROWS = NUM_LABELS + 1


def _body(tok_hbm, cu_hbm, tab_hbm, x_hbm, mask_hbm, ts_hbm,
          cu_v, tok_v, idx_v, mask_v, ts_v, buf, tab_v, zero_v,
          tsem, ssem0, ssem1):
    cid = lax.axis_index("c")
    sid = lax.axis_index("s")
    b = sid
    p0 = cid * ROWS_PER_W
    rbase = b * MAX_LEN + p0

    tab_cp = pltpu.async_copy(tab_hbm, tab_v, tsem)
    pltpu.sync_copy(cu_hbm, cu_v)

    iot = lax.broadcasted_iota(jnp.int32, (16,), 0)
    starts16 = cu_v[pl.ds(0, 16)]
    ends16 = plsc.load_gather(cu_v, [iot + 1])
    clipped16 = jnp.minimum(ends16 - starts16, MAX_LEN)

    @pl.when(jnp.logical_and(cid == 0, sid == 0))
    def _():
        ts = jnp.max(clipped16)
        ts_v[...] = jnp.broadcast_to(ts, (16,))
        pltpu.sync_copy(ts_v, ts_hbm)

    bvec = jnp.broadcast_to(b, (16,))
    s_splat = plsc.load_gather(cu_v, [bvec])
    e_splat = plsc.load_gather(cu_v, [bvec + 1])
    c_splat = jnp.minimum(e_splat - s_splat, MAX_LEN)
    s_scalar = jnp.max(s_splat)

    start = jnp.minimum(s_scalar + p0, TOTAL - TOK_BUF)
    start_al = pl.multiple_of(jnp.bitwise_and(start, -8), 8)
    pltpu.sync_copy(tok_hbm.at[pl.ds(start_al, TOK_BUF)], tok_v)

    for j in range(ROWS_PER_W // 16):
        p_vec = p0 + j * 16 + iot
        valid = p_vec < c_splat
        t_idx = jnp.clip(s_splat + p_vec - start_al, 0, TOK_BUF - 1)
        tok = plsc.load_gather(tok_v, [t_idx])
        g = jnp.where(valid, tok, PAD_ID)
        idx_v[pl.ds(j * 16, 16)] = g * EMB
        mask_v[pl.ds(j * 16, 16)] = jnp.where(valid, 1.0, 0.0)

    pltpu.sync_copy(mask_v, mask_hbm.at[pl.ds(rbase, ROWS_PER_W)])

    @plsc.parallel_loop(0, CHUNK, unroll=2)
    def zero_body(r):
        z = jnp.zeros((16,), jnp.float32)
        for k in range(EMB // 16):
            zero_v[r, pl.ds(k * 16, 16)] = z

    c_scalar = jnp.max(c_splat)
    nvalid = jnp.clip(c_scalar - p0, 0, ROWS_PER_W)
    tab_cp.wait()

    ssems = (ssem0, ssem1)

    @pl.loop(0, NCHUNK // NBUF)
    def group_body(g):
        c0 = g * NBUF
        for bsel in range(NBUF):
            c = c0 + bsel
            dst = x_hbm.at[pl.ds(rbase + c * CHUNK, CHUNK)]

            @pl.when(g > 0)
            def _(bsel=bsel, dst=dst):
                pltpu.make_async_copy(buf.at[bsel], dst, ssems[bsel]).wait()

            has_data = c * CHUNK < nvalid

            @pl.when(has_data)
            def _(c=c, bsel=bsel, dst=dst):
                @plsc.parallel_loop(0, CHUNK, unroll=3)
                def row_body(r):
                    base = plsc.load_gather(
                        idx_v, [jnp.broadcast_to(c * CHUNK + r, (16,))])
                    for k in range(EMB // 16):
                        v = plsc.load_gather(tab_v, [base + (k * 16) + iot])
                        buf[bsel, r, pl.ds(k * 16, 16)] = v

                pltpu.make_async_copy(buf.at[bsel], dst, ssems[bsel]).start()

            @pl.when(jnp.logical_not(has_data))
            def _(bsel=bsel, dst=dst):
                pltpu.make_async_copy(zero_v, dst, ssems[bsel]).start()

    for bsel in range(NBUF):
        c = NCHUNK - NBUF + bsel
        dst = x_hbm.at[pl.ds(rbase + c * CHUNK, CHUNK)]
        pltpu.make_async_copy(buf.at[bsel], dst, ssems[bsel]).wait()


@jax.jit
def _run(tokens, cu_pad, table_flat):
    mesh = plsc.VectorSubcoreMesh(core_axis_name="c", subcore_axis_name="s",
                                  num_cores=2, num_subcores=16)
    f = pl.kernel(
        _body,
        out_type=(
            jax.ShapeDtypeStruct((BATCH * MAX_LEN, EMB), jnp.float32),
            jax.ShapeDtypeStruct((BATCH * MAX_LEN,), jnp.float32),
            jax.ShapeDtypeStruct((16,), jnp.int32),
        ),
        mesh=mesh,
        compiler_params=pltpu.CompilerParams(needs_layout_passes=False),
        scratch_types=[
            pltpu.VMEM((24,), jnp.int32),
            pltpu.VMEM((TOK_BUF,), jnp.int32),
            pltpu.VMEM((ROWS_PER_W,), jnp.int32),
            pltpu.VMEM((ROWS_PER_W,), jnp.float32),
            pltpu.VMEM((16,), jnp.int32),
            pltpu.VMEM((NBUF, CHUNK, EMB), jnp.float32),
            pltpu.VMEM((TAB_ROWS * EMB,), jnp.float32),
            pltpu.VMEM((CHUNK, EMB), jnp.float32),
            pltpu.SemaphoreType.DMA,
            pltpu.SemaphoreType.DMA,
            pltpu.SemaphoreType.DMA,
        ],
    )
    return f(tokens, cu_pad, table_flat)


def kernel(tokens, cu_seqlens, embeddings):
    cu_pad = jnp.concatenate(
        [cu_seqlens.astype(jnp.int32),
         jnp.zeros((24 - cu_seqlens.shape[0],), jnp.int32)])
    table_flat = jnp.concatenate(
        [embeddings, jnp.zeros((1, EMB), embeddings.dtype)],
        axis=0).reshape(-1)
    x_flat, mask_flat, ts = _run(tokens, cu_pad, table_flat)
    x = x_flat.reshape(BATCH, MAX_LEN, EMB)
    mask = mask_flat.reshape(BATCH, MAX_LEN)
    return x, mask, ts[0]


# speedup vs baseline: 1.0714x; 1.0714x over previous
import functools

import jax
import jax.numpy as jnp
from jax import lax
from jax.experimental import pallas as pl
from jax.experimental.pallas import tpu as pltpu, tpu_sc as plsc

NUM_LABELS = 33
EMB = 512
MAX_LEN = 2048
BATCH = 16
TOTAL = 16384

ROWS_PER_W = 1024
CHUNK = 64
NCHUNK = ROWS_PER_W // CHUNK
NBUF = 2
TOK_BUF = 1032
PAD_ID = NUM_LABELS
TAB_ROWS = NUM_LABELS + 1


def _body(tok_hbm, cu_hbm, tab_hbm, x_hbm, mask_hbm, ts_hbm,
          cu_v, tok_v, idx_v, mask_v, ts_v, buf, tab_v, zero_v,
          tsem, ssem0, ssem1):
    cid = lax.axis_index("c")
    sid = lax.axis_index("s")
    b = sid
    p0 = cid * ROWS_PER_W
    rbase = b * MAX_LEN + p0

    tab_cp = pltpu.async_copy(tab_hbm, tab_v, tsem)
    pltpu.sync_copy(cu_hbm, cu_v)

    iot = lax.broadcasted_iota(jnp.int32, (16,), 0)
    starts16 = cu_v[pl.ds(0, 16)]
    ends16 = plsc.load_gather(cu_v, [iot + 1])
    clipped16 = jnp.minimum(ends16 - starts16, MAX_LEN)

    @pl.when(jnp.logical_and(cid == 0, sid == 0))
    def _():
        ts = jnp.max(clipped16)
        ts_v[...] = jnp.broadcast_to(ts, (16,))
        pltpu.sync_copy(ts_v, ts_hbm)

    bvec = jnp.broadcast_to(b, (16,))
    s_splat = plsc.load_gather(cu_v, [bvec])
    e_splat = plsc.load_gather(cu_v, [bvec + 1])
    c_splat = jnp.minimum(e_splat - s_splat, MAX_LEN)
    s_scalar = jnp.max(s_splat)

    start = jnp.minimum(s_scalar + p0, TOTAL - TOK_BUF)
    start_al = pl.multiple_of(jnp.bitwise_and(start, -8), 8)
    pltpu.sync_copy(tok_hbm.at[pl.ds(start_al, TOK_BUF)], tok_v)

    for j in range(ROWS_PER_W // 16):
        p_vec = p0 + j * 16 + iot
        valid = p_vec < c_splat
        t_idx = jnp.clip(s_splat + p_vec - start_al, 0, TOK_BUF - 1)
        tok = plsc.load_gather(tok_v, [t_idx])
        g = jnp.where(valid, tok, PAD_ID)
        idx_v[pl.ds(j * 16, 16)] = g * EMB
        mask_v[pl.ds(j * 16, 16)] = jnp.where(valid, 1.0, 0.0)

    pltpu.sync_copy(mask_v, mask_hbm.at[pl.ds(rbase, ROWS_PER_W)])

    @plsc.parallel_loop(0, CHUNK, unroll=2)
    def zero_body(r):
        z = jnp.zeros((16,), jnp.float32)
        for k in range(EMB // 16):
            zero_v[r, pl.ds(k * 16, 16)] = z

    c_scalar = jnp.max(c_splat)
    nvalid = jnp.clip(c_scalar - p0, 0, ROWS_PER_W)
    tab_cp.wait()

    ssems = (ssem0, ssem1)

    @pl.loop(0, NCHUNK // NBUF)
    def group_body(g):
        c0 = g * NBUF
        for bsel in range(NBUF):
            c = c0 + bsel
            dst = x_hbm.at[pl.ds(rbase + c * CHUNK, CHUNK)]

            @pl.when(g > 0)
            def _(bsel=bsel, dst=dst):
                pltpu.make_async_copy(buf.at[bsel], dst, ssems[bsel]).wait()

            has_data = c * CHUNK < nvalid

            @pl.when(has_data)
            def _(c=c, bsel=bsel, dst=dst):
                @plsc.parallel_loop(0, CHUNK, unroll=2)
                def row_body(r):
                    base = plsc.load_gather(
                        idx_v, [jnp.broadcast_to(c * CHUNK + r, (16,))])
                    for k in range(EMB // 16):
                        v = plsc.load_gather(tab_v, [base + (k * 16) + iot])
                        buf[bsel, r, pl.ds(k * 16, 16)] = v

                pltpu.make_async_copy(buf.at[bsel], dst, ssems[bsel]).start()

            @pl.when(jnp.logical_not(has_data))
            def _(bsel=bsel, dst=dst):
                pltpu.make_async_copy(zero_v, dst, ssems[bsel]).start()

    for bsel in range(NBUF):
        c = NCHUNK - NBUF + bsel
        dst = x_hbm.at[pl.ds(rbase + c * CHUNK, CHUNK)]
        pltpu.make_async_copy(buf.at[bsel], dst, ssems[bsel]).wait()


@jax.jit
def _run(tokens, cu_pad, table_flat):
    mesh = plsc.VectorSubcoreMesh(core_axis_name="c", subcore_axis_name="s",
                                  num_cores=2, num_subcores=16)
    f = pl.kernel(
        _body,
        out_type=(
            jax.ShapeDtypeStruct((BATCH * MAX_LEN, EMB), jnp.float32),
            jax.ShapeDtypeStruct((BATCH * MAX_LEN,), jnp.float32),
            jax.ShapeDtypeStruct((16,), jnp.int32),
        ),
        mesh=mesh,
        compiler_params=pltpu.CompilerParams(needs_layout_passes=False),
        scratch_types=[
            pltpu.VMEM((24,), jnp.int32),
            pltpu.VMEM((TOK_BUF,), jnp.int32),
            pltpu.VMEM((ROWS_PER_W,), jnp.int32),
            pltpu.VMEM((ROWS_PER_W,), jnp.float32),
            pltpu.VMEM((16,), jnp.int32),
            pltpu.VMEM((NBUF, CHUNK, EMB), jnp.float32),
            pltpu.VMEM((TAB_ROWS * EMB,), jnp.float32),
            pltpu.VMEM((CHUNK, EMB), jnp.float32),
            pltpu.SemaphoreType.DMA,
            pltpu.SemaphoreType.DMA,
            pltpu.SemaphoreType.DMA,
        ],
    )
    return f(tokens, cu_pad, table_flat)


def kernel(tokens, cu_seqlens, embeddings):
    cu_pad = jnp.concatenate(
        [cu_seqlens.astype(jnp.int32),
         jnp.zeros((24 - cu_seqlens.shape[0],), jnp.int32)])
    table_flat = jnp.concatenate(
        [embeddings, jnp.zeros((1, EMB), embeddings.dtype)],
        axis=0).reshape(-1)
    x_flat, mask_flat, ts = _run(tokens, cu_pad, table_flat)
    x = x_flat.reshape(BATCH, MAX_LEN, EMB)
    mask = mask_flat.reshape(BATCH, MAX_LEN)
    return x, mask, ts[0]
